# Initial kernel scaffold; baseline (speedup 1.0000x reference)
import functools

import jax
import jax.numpy as jnp
from jax.experimental import pallas as pl
from jax.experimental.pallas import tpu as pltpu

H = 1024
FF = 1408
E = 8
K = 2
T = 2048
BT = 256
NT = T // BT


def _moe_body(x_ref, gate_ref, bias_ref, sw1_ref, sw2_ref, sw3_ref,
              w1_ref, w2_ref, w3_ref, out_ref, acc_ref, comb_ref):
    s = pl.program_id(0)
    t = pl.program_id(1)

    @pl.when((s == 0) & (t == 0))
    def _router():
        x = x_ref[:]
        logits = jax.lax.dot_general(
            x, gate_ref[:], (((1,), (1,)), ((), ())),
            precision=jax.lax.Precision.HIGHEST,
            preferred_element_type=jnp.float32)
        scores = jax.nn.sigmoid(logits)
        routing = scores + bias_ref[:]
        iota = jax.lax.broadcasted_iota(jnp.int32, (T, E), 1)
        m1 = jnp.max(routing, axis=1, keepdims=True)
        i1 = jnp.min(jnp.where(routing == m1, iota, E), axis=1, keepdims=True)
        mask1 = iota == i1
        routing2 = jnp.where(mask1, -jnp.inf, routing)
        m2 = jnp.max(routing2, axis=1, keepdims=True)
        i2 = jnp.min(jnp.where(routing2 == m2, iota, E), axis=1, keepdims=True)
        mask2 = iota == i2
        s1 = jnp.sum(jnp.where(mask1, scores, 0.0), axis=1, keepdims=True)
        s2 = jnp.sum(jnp.where(mask2, scores, 0.0), axis=1, keepdims=True)
        denom = s1 + s2
        comb = (jnp.where(mask1, s1, 0.0) + jnp.where(mask2, s2, 0.0)) / denom
        comb_ref[:] = comb

    xt = x_ref[pl.ds(t * BT, BT), :].astype(jnp.bfloat16)

    def ffn(w1, w3, w2):
        h1 = jax.lax.dot_general(xt, w1, (((1,), (1,)), ((), ())),
                                 preferred_element_type=jnp.float32)
        h3 = jax.lax.dot_general(xt, w3, (((1,), (1,)), ((), ())),
                                 preferred_element_type=jnp.float32)
        h = (jax.nn.silu(h1) * h3).astype(jnp.bfloat16)
        return jax.lax.dot_general(h, w2, (((1,), (1,)), ((), ())),
                                   preferred_element_type=jnp.float32)

    @pl.when(s == 0)
    def _shared():
        acc_ref[pl.ds(t * BT, BT), :] = ffn(sw1_ref[:], sw3_ref[:], sw2_ref[:])

    @pl.when(s > 0)
    def _expert():
        o = ffn(w1_ref[0], w3_ref[0], w2_ref[0])
        g = comb_ref[pl.ds(t * BT, BT), pl.ds(s - 1, 1)]
        acc_ref[pl.ds(t * BT, BT), :] += g * o

    @pl.when(s == E)
    def _write():
        out_ref[:] = acc_ref[pl.ds(t * BT, BT), :]


@jax.jit
def kernel(hidden_states, gate_w, bias, sw1, sw2, sw3, W1, W2, W3):
    orig_shape = hidden_states.shape
    x = hidden_states.reshape(T, H)
    w1b = W1.astype(jnp.bfloat16)
    w2b = W2.astype(jnp.bfloat16)
    w3b = W3.astype(jnp.bfloat16)
    sw1b = sw1[0].astype(jnp.bfloat16)
    sw2b = sw2[0].astype(jnp.bfloat16)
    sw3b = sw3[0].astype(jnp.bfloat16)
    bias2 = bias.reshape(1, E)

    const = lambda *dims: pl.BlockSpec(dims, lambda s, t: (0,) * len(dims))
    out = pl.pallas_call(
        _moe_body,
        grid=(E + 1, NT),
        in_specs=[
            const(T, H),
            const(E, H),
            const(1, E),
            const(FF, H), const(H, FF), const(FF, H),
            pl.BlockSpec((1, FF, H), lambda s, t: (jnp.maximum(s - 1, 0), 0, 0)),
            pl.BlockSpec((1, H, FF), lambda s, t: (jnp.maximum(s - 1, 0), 0, 0)),
            pl.BlockSpec((1, FF, H), lambda s, t: (jnp.maximum(s - 1, 0), 0, 0)),
        ],
        out_specs=pl.BlockSpec((BT, H), lambda s, t: (t, 0)),
        out_shape=jax.ShapeDtypeStruct((T, H), jnp.float32),
        scratch_shapes=[
            pltpu.VMEM((T, H), jnp.float32),
            pltpu.VMEM((T, E), jnp.float32),
        ],
        compiler_params=pltpu.CompilerParams(
            dimension_semantics=("arbitrary", "arbitrary"),
        ),
    )(x, gate_w, bias2, sw1b, sw2b, sw3b, w1b, w2b, w3b)
    return out.reshape(orig_shape)


# speedup vs baseline: 1.0321x; 1.0321x over previous
import functools

import jax
import jax.numpy as jnp
from jax.experimental import pallas as pl
from jax.experimental.pallas import tpu as pltpu

H = 1024
FF = 1408
E = 8
K = 2
T = 2048
BT = 256
NT = T // BT


def _moe_body(x_ref, gate_ref, bias_ref, sw1_ref, sw2_ref, sw3_ref,
              w1_ref, w2_ref, w3_ref, out_ref, acc_ref, comb_ref):
    s = pl.program_id(0)
    t = pl.program_id(1)

    @pl.when((s == 0) & (t == 0))
    def _router():
        x = x_ref[:].astype(jnp.bfloat16)
        logits = jax.lax.dot_general(
            x, gate_ref[:].astype(jnp.bfloat16), (((1,), (1,)), ((), ())),
            preferred_element_type=jnp.float32)
        scores = jax.nn.sigmoid(logits)
        routing = scores + bias_ref[:]
        iota = jax.lax.broadcasted_iota(jnp.int32, (T, E), 1)
        m1 = jnp.max(routing, axis=1, keepdims=True)
        i1 = jnp.min(jnp.where(routing == m1, iota, E), axis=1, keepdims=True)
        mask1 = iota == i1
        routing2 = jnp.where(mask1, -jnp.inf, routing)
        m2 = jnp.max(routing2, axis=1, keepdims=True)
        i2 = jnp.min(jnp.where(routing2 == m2, iota, E), axis=1, keepdims=True)
        mask2 = iota == i2
        s1 = jnp.sum(jnp.where(mask1, scores, 0.0), axis=1, keepdims=True)
        s2 = jnp.sum(jnp.where(mask2, scores, 0.0), axis=1, keepdims=True)
        denom = s1 + s2
        comb = (jnp.where(mask1, s1, 0.0) + jnp.where(mask2, s2, 0.0)) / denom
        comb_ref[:] = comb

    xt = x_ref[pl.ds(t * BT, BT), :].astype(jnp.bfloat16)

    def ffn(w1, w3, w2):
        h1 = jax.lax.dot_general(xt, w1, (((1,), (1,)), ((), ())),
                                 preferred_element_type=jnp.float32)
        h3 = jax.lax.dot_general(xt, w3, (((1,), (1,)), ((), ())),
                                 preferred_element_type=jnp.float32)
        h = (jax.nn.silu(h1) * h3).astype(jnp.bfloat16)
        return jax.lax.dot_general(h, w2, (((1,), (1,)), ((), ())),
                                   preferred_element_type=jnp.float32)

    @pl.when(s == 0)
    def _shared():
        acc_ref[pl.ds(t * BT, BT), :] = ffn(sw1_ref[:], sw3_ref[:], sw2_ref[:])

    @pl.when(s > 0)
    def _expert():
        o = ffn(w1_ref[0], w3_ref[0], w2_ref[0])
        ct = comb_ref[pl.ds(t * BT, BT), :]
        lane = jax.lax.broadcasted_iota(jnp.int32, (BT, E), 1)
        g = jnp.sum(jnp.where(lane == s - 1, ct, 0.0), axis=1, keepdims=True)
        acc_ref[pl.ds(t * BT, BT), :] += g * o

    @pl.when(s == E)
    def _write():
        out_ref[:] = acc_ref[pl.ds(t * BT, BT), :]


@jax.jit
def kernel(hidden_states, gate_w, bias, sw1, sw2, sw3, W1, W2, W3):
    orig_shape = hidden_states.shape
    x = hidden_states.reshape(T, H)
    w1b = W1.astype(jnp.bfloat16)
    w2b = W2.astype(jnp.bfloat16)
    w3b = W3.astype(jnp.bfloat16)
    sw1b = sw1[0].astype(jnp.bfloat16)
    sw2b = sw2[0].astype(jnp.bfloat16)
    sw3b = sw3[0].astype(jnp.bfloat16)
    bias2 = bias.reshape(1, E)

    const = lambda *dims: pl.BlockSpec(dims, lambda s, t: (0,) * len(dims))
    out = pl.pallas_call(
        _moe_body,
        grid=(E + 1, NT),
        in_specs=[
            const(T, H),
            const(E, H),
            const(1, E),
            const(FF, H), const(H, FF), const(FF, H),
            pl.BlockSpec((1, FF, H), lambda s, t: (jnp.maximum(s - 1, 0), 0, 0)),
            pl.BlockSpec((1, H, FF), lambda s, t: (jnp.maximum(s - 1, 0), 0, 0)),
            pl.BlockSpec((1, FF, H), lambda s, t: (jnp.maximum(s - 1, 0), 0, 0)),
        ],
        out_specs=pl.BlockSpec((BT, H), lambda s, t: (t, 0)),
        out_shape=jax.ShapeDtypeStruct((T, H), jnp.float32),
        scratch_shapes=[
            pltpu.VMEM((T, H), jnp.float32),
            pltpu.VMEM((T, E), jnp.float32),
        ],
        compiler_params=pltpu.CompilerParams(
            dimension_semantics=("arbitrary", "arbitrary"),
            vmem_limit_bytes=64 * 1024 * 1024,
        ),
    )(x, gate_w, bias2, sw1b, sw2b, sw3b, w1b, w2b, w3b)
    return out.reshape(orig_shape)
